# baseline (speedup 1.0000x reference)
import functools

import jax
import jax.numpy as jnp
from jax import lax
from jax.experimental import pallas as pl
from jax.experimental.pallas import tpu as pltpu
from jax.experimental.pallas import tpu_sc as plsc

_NC = 32
_K = 16
_LEVELS = 4
_LANES = 16
_WORKERS = 32
_CHUNK = 16


def _sc_encode_body(d, n, x_hbm, cols_hbm, svp_hbm, out_hbm, xbuf0, xbuf1,
                    colbuf, svpbuf, cbuf, sem0, sem1):
    wid = lax.axis_index("s") * 2 + lax.axis_index("c")
    rows_per_w = n // _WORKERS
    nchunk = rows_per_w // _CHUNK
    pltpu.sync_copy(cols_hbm, colbuf)
    pltpu.sync_copy(svp_hbm, svpbuf)
    base_w = wid * rows_per_w
    bufs = (xbuf0, xbuf1)
    sems = (sem0, sem1)

    for b in range(2):
        pltpu.async_copy(
            x_hbm.at[pl.ds(base_w + b * _CHUNK, _CHUNK), :], bufs[b], sems[b])

    def process_chunk(ci, xbuf):
        def row_body(r, carry2):
            row_splat = jnp.full((_LANES,), 0, jnp.int32) + r
            for h in range(_NC // _LANES):
                i_vec = lax.iota(jnp.int32, _LANES) + _LANES * h
                g = jnp.zeros((_LANES,), jnp.int32)
                for level in range(_LEVELS):
                    cols = colbuf[pl.ds(_NC * level + _LANES * h, _LANES)]
                    vals = plsc.load_gather(xbuf, [row_splat, cols])
                    tidx = level * (8 * _NC) + g * _NC + i_vec
                    thr = plsc.load_gather(svpbuf, [tidx])
                    g = g * 2 + (vals > thr).astype(jnp.int32)
                cbuf[r, pl.ds(_LANES * h, _LANES)] = g
            return carry2

        lax.fori_loop(0, _CHUNK, row_body, 0)
        base = base_w + ci * _CHUNK
        pltpu.sync_copy(cbuf, out_hbm.at[pl.ds(base, _CHUNK), :])

    def pair_body(pi, carry):
        for b in range(2):
            ci = pi * 2 + b
            base = base_w + ci * _CHUNK
            pltpu.make_async_copy(
                x_hbm.at[pl.ds(base, _CHUNK), :], bufs[b], sems[b]).wait()
            process_chunk(ci, bufs[b])
            nxt = ci + 2

            @pl.when(nxt < nchunk)
            def _():
                pltpu.async_copy(
                    x_hbm.at[pl.ds(base_w + nxt * _CHUNK, _CHUNK), :],
                    bufs[b], sems[b])
        return carry

    lax.fori_loop(0, nchunk // 2, pair_body, 0)


def _tc_body(codes_ref, lut_ref, bias_ref, out_ref):
    rows = codes_ref.shape[0]
    nc = _NC
    g = codes_ref[...]

    c = nc * _K
    exp_i = jax.lax.broadcasted_iota(jnp.int32, (nc, c), 0)
    exp_l = jax.lax.broadcasted_iota(jnp.int32, (nc, c), 1)
    expand = (exp_i == (exp_l // _K)).astype(jnp.bfloat16)
    e = jax.lax.dot_general(
        g.astype(jnp.bfloat16), expand, (((1,), (0,)), ((), ())),
        preferred_element_type=jnp.float32)
    lane_code = (jax.lax.broadcasted_iota(jnp.int32, (rows, c), 1)
                 % _K).astype(jnp.float32)
    oh = (e == lane_code).astype(jnp.bfloat16)

    acc = jax.lax.dot_general(
        oh, lut_ref[...], (((1,), (0,)), ((), ())),
        preferred_element_type=jnp.float32)
    out_ref[...] = acc + bias_ref[...]


@functools.partial(jax.jit, static_argnames=())
def kernel(x, split_idxs, split_vals, lookup_tables, bias):
    n, d = x.shape
    nc, k, out_f = lookup_tables.shape
    sub = d // nc

    col_tab = (split_idxs.T.astype(jnp.int32)
               + sub * jnp.arange(nc, dtype=jnp.int32)[None, :]).reshape(-1)
    svp_flat = split_vals.transpose(1, 2, 0).reshape(-1)
    lut_flat = lookup_tables.reshape(nc * k, out_f).astype(jnp.bfloat16)
    bias2 = bias.reshape(1, out_f)

    sc_encode = functools.partial(
        pl.kernel,
        mesh=plsc.VectorSubcoreMesh(core_axis_name="c", subcore_axis_name="s"),
        out_type=jax.ShapeDtypeStruct((n, nc), jnp.int32),
        scratch_types=[
            pltpu.VMEM((_CHUNK, d), jnp.float32),
            pltpu.VMEM((_CHUNK, d), jnp.float32),
            pltpu.VMEM((_LEVELS * nc,), jnp.int32),
            pltpu.VMEM((_LEVELS * 8 * nc,), jnp.float32),
            pltpu.VMEM((_CHUNK, nc), jnp.int32),
            pltpu.SemaphoreType.DMA,
            pltpu.SemaphoreType.DMA,
        ],
        compiler_params=pltpu.CompilerParams(
            needs_layout_passes=False, use_tc_tiling_on_sc=True),
    )(functools.partial(_sc_encode_body, d, n))
    codes = sc_encode(x, col_tab, svp_flat)

    rows_blk = 256
    nb = n // rows_blk

    return pl.pallas_call(
        _tc_body,
        grid=(nb,),
        in_specs=[
            pl.BlockSpec((rows_blk, nc), lambda i: (i, 0)),
            pl.BlockSpec((nc * k, out_f), lambda i: (0, 0)),
            pl.BlockSpec((1, out_f), lambda i: (0, 0)),
        ],
        out_specs=pl.BlockSpec((rows_blk, out_f), lambda i: (i, 0)),
        out_shape=jax.ShapeDtypeStruct((n, out_f), jnp.float32),
        compiler_params=pltpu.CompilerParams(
            dimension_semantics=("arbitrary",)),
    )(codes, lut_flat, bias2)


# speedup vs baseline: 1.0956x; 1.0956x over previous
import functools

import jax
import jax.numpy as jnp
from jax import lax
from jax.experimental import pallas as pl
from jax.experimental.pallas import tpu as pltpu
from jax.experimental.pallas import tpu_sc as plsc

_NC = 32
_K = 16
_LEVELS = 4
_LANES = 16
_WORKERS = 32
_CHUNK = 16


def _sc_encode_body(d, n, x_hbm, cols_hbm, svp_hbm, out_hbm, xbuf0, xbuf1,
                    colbuf, svpbuf, cbuf, sem0, sem1):
    wid = lax.axis_index("s") * 2 + lax.axis_index("c")
    rows_per_w = n // _WORKERS
    nchunk = rows_per_w // _CHUNK
    pltpu.sync_copy(cols_hbm, colbuf)
    pltpu.sync_copy(svp_hbm, svpbuf)
    base_w = wid * rows_per_w
    bufs = (xbuf0, xbuf1)
    sems = (sem0, sem1)

    for b in range(2):
        pltpu.async_copy(
            x_hbm.at[pl.ds(base_w + b * _CHUNK, _CHUNK), :], bufs[b], sems[b])

    def process_chunk(ci, xbuf):
        def row_body(r, carry2):
            row_splat = jnp.full((_LANES,), 0, jnp.int32) + r
            for h in range(_NC // _LANES):
                i_vec = lax.iota(jnp.int32, _LANES) + _LANES * h
                g = jnp.zeros((_LANES,), jnp.int32)
                for level in range(_LEVELS):
                    cols = colbuf[pl.ds(_NC * level + _LANES * h, _LANES)]
                    vals = plsc.load_gather(xbuf, [row_splat, cols])
                    tidx = level * (8 * _NC) + g * _NC + i_vec
                    thr = plsc.load_gather(svpbuf, [tidx])
                    g = g * 2 + (vals > thr).astype(jnp.int32)
                cbuf[r, pl.ds(_LANES * h, _LANES)] = g
            return carry2

        lax.fori_loop(0, _CHUNK, row_body, 0)
        base = base_w + ci * _CHUNK
        pltpu.sync_copy(cbuf, out_hbm.at[pl.ds(base, _CHUNK), :])

    def pair_body(pi, carry):
        for b in range(2):
            ci = pi * 2 + b
            base = base_w + ci * _CHUNK
            pltpu.make_async_copy(
                x_hbm.at[pl.ds(base, _CHUNK), :], bufs[b], sems[b]).wait()
            process_chunk(ci, bufs[b])
            nxt = ci + 2

            @pl.when(nxt < nchunk)
            def _():
                pltpu.async_copy(
                    x_hbm.at[pl.ds(base_w + nxt * _CHUNK, _CHUNK), :],
                    bufs[b], sems[b])
        return carry

    lax.fori_loop(0, nchunk // 2, pair_body, 0)


def _tc_body(codes_ref, lut_ref, bias_ref, out_ref):
    rows = codes_ref.shape[0]
    nc = _NC
    g = codes_ref[...]

    c = nc * _K
    exp_i = jax.lax.broadcasted_iota(jnp.int32, (nc, c), 0)
    exp_l = jax.lax.broadcasted_iota(jnp.int32, (nc, c), 1)
    expand = (exp_i == (exp_l // _K)).astype(jnp.bfloat16)
    e = jax.lax.dot_general(
        g.astype(jnp.bfloat16), expand, (((1,), (0,)), ((), ())),
        preferred_element_type=jnp.float32)
    lane_code = (jax.lax.broadcasted_iota(jnp.int32, (rows, c), 1)
                 % _K).astype(jnp.float32)
    oh = (e == lane_code).astype(jnp.bfloat16)

    acc = jax.lax.dot_general(
        oh, lut_ref[...], (((1,), (0,)), ((), ())),
        preferred_element_type=jnp.float32)
    out_ref[...] = acc + bias_ref[...]


@functools.partial(jax.jit, static_argnames=())
def kernel(x, split_idxs, split_vals, lookup_tables, bias):
    n, d = x.shape
    nc, k, out_f = lookup_tables.shape
    sub = d // nc

    col_tab = (split_idxs.T.astype(jnp.int32)
               + sub * jnp.arange(nc, dtype=jnp.int32)[None, :]).reshape(-1)
    svp_flat = split_vals.transpose(1, 2, 0).reshape(-1)
    lut_flat = lookup_tables.reshape(nc * k, out_f).astype(jnp.bfloat16)
    bias2 = bias.reshape(1, out_f)

    sc_encode = functools.partial(
        pl.kernel,
        mesh=plsc.VectorSubcoreMesh(core_axis_name="c", subcore_axis_name="s"),
        out_type=jax.ShapeDtypeStruct((n, nc), jnp.int32),
        scratch_types=[
            pltpu.VMEM((_CHUNK, d), jnp.float32),
            pltpu.VMEM((_CHUNK, d), jnp.float32),
            pltpu.VMEM((_LEVELS * nc,), jnp.int32),
            pltpu.VMEM((_LEVELS * 8 * nc,), jnp.float32),
            pltpu.VMEM((_CHUNK, nc), jnp.int32),
            pltpu.SemaphoreType.DMA,
            pltpu.SemaphoreType.DMA,
        ],
        compiler_params=pltpu.CompilerParams(
            needs_layout_passes=False, use_tc_tiling_on_sc=True),
    )(functools.partial(_sc_encode_body, d, n))
    codes = sc_encode(x, col_tab, svp_flat)

    rows_blk = 1024
    nb = n // rows_blk

    return pl.pallas_call(
        _tc_body,
        grid=(nb,),
        in_specs=[
            pl.BlockSpec((rows_blk, nc), lambda i: (i, 0)),
            pl.BlockSpec((nc * k, out_f), lambda i: (0, 0)),
            pl.BlockSpec((1, out_f), lambda i: (0, 0)),
        ],
        out_specs=pl.BlockSpec((rows_blk, out_f), lambda i: (i, 0)),
        out_shape=jax.ShapeDtypeStruct((n, out_f), jnp.float32),
        compiler_params=pltpu.CompilerParams(
            dimension_semantics=("arbitrary",)),
    )(codes, lut_flat, bias2)
